# baseline (speedup 1.0000x reference)
import jax
import jax.numpy as jnp
from jax import lax
from jax.experimental import pallas as pl
from jax.experimental.pallas import tpu as pltpu
from jax.experimental.pallas import tpu_sc as plsc

NUM_GRAPHS = 512
IN_DIM = 256
N_NODES = 50000

NC, NS = 2, 16
NW = NC * NS
SEGS_PER_W = NUM_GRAPHS // NW
C = 128
NSEG_PAD = 528
NVEC = IN_DIM // 16


def _sc_body(node_hbm, segs_hbm, sums_hbm, cnts_hbm,
             rows_v, segs_v, sums_v, cnts_v):
    c = lax.axis_index("c")
    s = lax.axis_index("s")
    wid = s * NC + c
    seg0 = wid * SEGS_PER_W

    pltpu.sync_copy(segs_hbm, segs_v)

    def do_segment(t, carry):
        seg_pair = segs_v[pl.ds(seg0 + t, 16)]
        a = seg_pair[0]
        b = seg_pair[1]
        cnt = b - a
        w0 = (a // 8) * 8
        nch = (b - w0 + C - 1) // C

        def do_chunk(ch, acc):
            wb = w0 + ch * C
            cbase = jnp.minimum(wb, N_NODES - C)
            pltpu.sync_copy(node_hbm.at[pl.ds(cbase, C)], rows_v)
            p = jnp.maximum(a, wb)
            q = jnp.minimum(wb + C, b)

            def do_row(r, acc2):
                lidx = r - cbase
                return tuple(
                    acc2[k] + rows_v[lidx, pl.ds(k * 16, 16)]
                    for k in range(NVEC)
                )

            return lax.fori_loop(p, q, do_row, acc)

        zero16 = jnp.zeros((16,), jnp.float32)
        acc = lax.fori_loop(0, nch, do_chunk, (zero16,) * NVEC)
        for k in range(NVEC):
            sums_v[t, pl.ds(k * 16, 16)] = acc[k]
        cnts_v[t, pl.ds(0, 16)] = lax.broadcast_in_dim(
            cnt.astype(jnp.float32), (16,), ())
        return carry

    lax.fori_loop(0, SEGS_PER_W, do_segment, 0)

    pltpu.sync_copy(sums_v, sums_hbm.at[pl.ds(seg0, SEGS_PER_W)])
    pltpu.sync_copy(cnts_v, cnts_hbm.at[pl.ds(seg0, SEGS_PER_W)])


_sc_segment_sum = pl.kernel(
    _sc_body,
    out_type=[
        jax.ShapeDtypeStruct((NUM_GRAPHS, IN_DIM), jnp.float32),
        jax.ShapeDtypeStruct((NUM_GRAPHS, 16), jnp.float32),
    ],
    mesh=plsc.VectorSubcoreMesh(core_axis_name="c", subcore_axis_name="s",
                                num_cores=NC, num_subcores=NS),
    scratch_types=[
        pltpu.VMEM((C, IN_DIM), jnp.float32),
        pltpu.VMEM((NSEG_PAD,), jnp.int32),
        pltpu.VMEM((SEGS_PER_W, IN_DIM), jnp.float32),
        pltpu.VMEM((SEGS_PER_W, 16), jnp.float32),
    ],
)


def _mlp_body(sums_ref, cnts_ref, w1_ref, b1_ref, w2_ref, b2_ref,
              g_ref, z_ref):
    counts = cnts_ref[:, 0]
    g = sums_ref[...] / jnp.maximum(counts, 1.0)[:, None]
    g_ref[...] = g
    h = lax.dot_general(g, w1_ref[...], (((1,), (1,)), ((), ())),
                        preferred_element_type=jnp.float32) + b1_ref[0, :]
    h = jnp.maximum(h, 0.0)
    z_ref[...] = lax.dot_general(h, w2_ref[...], (((1,), (1,)), ((), ())),
                                 preferred_element_type=jnp.float32) + b2_ref[0, :]


@jax.jit
def kernel(node_rep, batch_ids, W1, b1, W2, b2):
    ids = batch_ids.astype(jnp.int32)
    seg_start = jnp.searchsorted(
        ids, jnp.arange(NSEG_PAD, dtype=jnp.int32), side="left"
    ).astype(jnp.int32)

    sums, cnts = _sc_segment_sum(node_rep, seg_start)

    out = pl.pallas_call(
        _mlp_body,
        in_specs=[
            pl.BlockSpec((NUM_GRAPHS, IN_DIM), lambda: (0, 0)),
            pl.BlockSpec((NUM_GRAPHS, 16), lambda: (0, 0)),
            pl.BlockSpec((IN_DIM, IN_DIM), lambda: (0, 0)),
            pl.BlockSpec((1, IN_DIM), lambda: (0, 0)),
            pl.BlockSpec((IN_DIM, IN_DIM), lambda: (0, 0)),
            pl.BlockSpec((1, IN_DIM), lambda: (0, 0)),
        ],
        out_specs=[
            pl.BlockSpec((NUM_GRAPHS, IN_DIM), lambda: (0, 0)),
            pl.BlockSpec((NUM_GRAPHS, IN_DIM), lambda: (0, 0)),
        ],
        out_shape=[
            jax.ShapeDtypeStruct((NUM_GRAPHS, IN_DIM), jnp.float32),
            jax.ShapeDtypeStruct((NUM_GRAPHS, IN_DIM), jnp.float32),
        ],
    )(sums, cnts, W1, b1.reshape(1, IN_DIM), W2, b2.reshape(1, IN_DIM))
    return (out[0], out[1])


# speedup vs baseline: 1.7648x; 1.7648x over previous
import jax
import jax.numpy as jnp
from jax import lax
from jax.experimental import pallas as pl
from jax.experimental.pallas import tpu as pltpu
from jax.experimental.pallas import tpu_sc as plsc

NUM_GRAPHS = 512
IN_DIM = 256
N_NODES = 50000

NC, NS = 2, 16
NW = NC * NS
SEGS_PER_W = NUM_GRAPHS // NW
C = 128
NIDS_PAD = N_NODES + 16
NVEC = IN_DIM // 16


def _sc_body(node_hbm, ids_hbm, sums_hbm, cnts_hbm,
             rows_v, ids_v, sums_v, cnts_v, segs_sm):
    c = lax.axis_index("c")
    s = lax.axis_index("s")
    wid = s * NC + c
    seg0 = wid * SEGS_PER_W

    pltpu.sync_copy(ids_hbm, ids_v)

    def find_boundary(t, carry):
        target = seg0 + t

        def probe(_, state):
            lo, hi = state
            mid = (lo + hi) // 2
            val = ids_v[pl.ds(mid, 16)][0]
            lt = val < target
            return (jnp.where(lt, mid + 1, lo), jnp.where(lt, hi, mid))

        lo, _ = lax.fori_loop(0, 16, probe,
                              (jnp.int32(0), jnp.int32(N_NODES)))
        segs_sm[t] = lo
        return carry

    lax.fori_loop(0, SEGS_PER_W + 1, find_boundary, 0)

    def do_segment(t, carry):
        a = segs_sm[t]
        b = segs_sm[t + 1]
        cnt = b - a
        w0 = (a // 8) * 8
        nch = (b - w0 + C - 1) // C

        def do_chunk(ch, acc):
            wb = w0 + ch * C
            cbase = jnp.minimum(wb, N_NODES - C)
            pltpu.sync_copy(node_hbm.at[pl.ds(cbase, C)], rows_v)
            p = jnp.maximum(a, wb)
            q = jnp.minimum(wb + C, b)

            def do_row(r, acc2):
                lidx = r - cbase
                return tuple(
                    acc2[k] + rows_v[lidx, pl.ds(k * 16, 16)]
                    for k in range(NVEC)
                )

            return lax.fori_loop(p, q, do_row, acc)

        zero16 = jnp.zeros((16,), jnp.float32)
        acc = lax.fori_loop(0, nch, do_chunk, (zero16,) * NVEC)
        for k in range(NVEC):
            sums_v[t, pl.ds(k * 16, 16)] = acc[k]
        cnts_v[t, pl.ds(0, 16)] = lax.broadcast_in_dim(
            cnt.astype(jnp.float32), (16,), ())
        return carry

    lax.fori_loop(0, SEGS_PER_W, do_segment, 0)

    pltpu.sync_copy(sums_v, sums_hbm.at[pl.ds(seg0, SEGS_PER_W)])
    pltpu.sync_copy(cnts_v, cnts_hbm.at[pl.ds(seg0, SEGS_PER_W)])


_sc_segment_sum = pl.kernel(
    _sc_body,
    out_type=[
        jax.ShapeDtypeStruct((NUM_GRAPHS, IN_DIM), jnp.float32),
        jax.ShapeDtypeStruct((NUM_GRAPHS, 16), jnp.float32),
    ],
    mesh=plsc.VectorSubcoreMesh(core_axis_name="c", subcore_axis_name="s",
                                num_cores=NC, num_subcores=NS),
    scratch_types=[
        pltpu.VMEM((C, IN_DIM), jnp.float32),
        pltpu.VMEM((NIDS_PAD,), jnp.int32),
        pltpu.VMEM((SEGS_PER_W, IN_DIM), jnp.float32),
        pltpu.VMEM((SEGS_PER_W, 16), jnp.float32),
        pltpu.SMEM((SEGS_PER_W + 1,), jnp.int32),
    ],
)


def _mlp_body(sums_ref, cnts_ref, w1_ref, b1_ref, w2_ref, b2_ref,
              g_ref, z_ref):
    counts = cnts_ref[:, 0]
    g = sums_ref[...] / jnp.maximum(counts, 1.0)[:, None]
    g_ref[...] = g
    h = lax.dot_general(g, w1_ref[...], (((1,), (1,)), ((), ())),
                        preferred_element_type=jnp.float32) + b1_ref[0, :]
    h = jnp.maximum(h, 0.0)
    z_ref[...] = lax.dot_general(h, w2_ref[...], (((1,), (1,)), ((), ())),
                                 preferred_element_type=jnp.float32) + b2_ref[0, :]


@jax.jit
def kernel(node_rep, batch_ids, W1, b1, W2, b2):
    ids = batch_ids.astype(jnp.int32)
    ids_pad = jnp.concatenate(
        [ids, jnp.full((NIDS_PAD - N_NODES,), NUM_GRAPHS, dtype=jnp.int32)])

    sums, cnts = _sc_segment_sum(node_rep, ids_pad)

    out = pl.pallas_call(
        _mlp_body,
        in_specs=[
            pl.BlockSpec((NUM_GRAPHS, IN_DIM), lambda: (0, 0)),
            pl.BlockSpec((NUM_GRAPHS, 16), lambda: (0, 0)),
            pl.BlockSpec((IN_DIM, IN_DIM), lambda: (0, 0)),
            pl.BlockSpec((1, IN_DIM), lambda: (0, 0)),
            pl.BlockSpec((IN_DIM, IN_DIM), lambda: (0, 0)),
            pl.BlockSpec((1, IN_DIM), lambda: (0, 0)),
        ],
        out_specs=[
            pl.BlockSpec((NUM_GRAPHS, IN_DIM), lambda: (0, 0)),
            pl.BlockSpec((NUM_GRAPHS, IN_DIM), lambda: (0, 0)),
        ],
        out_shape=[
            jax.ShapeDtypeStruct((NUM_GRAPHS, IN_DIM), jnp.float32),
            jax.ShapeDtypeStruct((NUM_GRAPHS, IN_DIM), jnp.float32),
        ],
    )(sums, cnts, W1, b1.reshape(1, IN_DIM), W2, b2.reshape(1, IN_DIM))
    return (out[0], out[1])


# speedup vs baseline: 2.0594x; 1.1670x over previous
import jax
import jax.numpy as jnp
from jax import lax
from jax.experimental import pallas as pl
from jax.experimental.pallas import tpu as pltpu
from jax.experimental.pallas import tpu_sc as plsc

NUM_GRAPHS = 512
IN_DIM = 256
N_NODES = 50000

NC, NS = 2, 16
NW = NC * NS
SEGS_PER_W = NUM_GRAPHS // NW
C = 128
NIDS_PAD = N_NODES + 16
NVEC = IN_DIM // 16


def _sc_body(node_hbm, ids_hbm, sums_hbm, cnts_hbm,
             rows0_v, rows1_v, ids_v, sums_v, cnts_v, segs_sm, plan_sm,
             sem0, sem1):
    c = lax.axis_index("c")
    s = lax.axis_index("s")
    wid = s * NC + c
    seg0 = wid * SEGS_PER_W

    pltpu.sync_copy(ids_hbm, ids_v)

    def find_boundary(t, carry):
        target = seg0 + t

        def probe(_, state):
            lo, hi = state
            mid = (lo + hi) // 2
            val = ids_v[pl.ds(mid, 16)][0]
            lt = val < target
            return (jnp.where(lt, mid + 1, lo), jnp.where(lt, hi, mid))

        lo, _ = lax.fori_loop(0, 16, probe,
                              (jnp.int32(0), jnp.int32(N_NODES)))
        segs_sm[t] = lo
        return carry

    lax.fori_loop(0, SEGS_PER_W + 1, find_boundary, 0)

    zero16 = jnp.zeros((16,), jnp.float32)

    def zero_vec(i, carry):
        sums_v[pl.ds(i * 16, 16)] = zero16
        return carry

    lax.fori_loop(0, (SEGS_PER_W + 1) * NVEC, zero_vec, 0)

    def init_cnt(t, carry):
        cnt = segs_sm[t + 1] - segs_sm[t]
        cnts_v[t, pl.ds(0, 16)] = lax.broadcast_in_dim(
            cnt.astype(jnp.float32), (16,), ())
        return carry

    lax.fori_loop(0, SEGS_PER_W, init_cnt, 0)

    def plan_segment(t, n):
        a = segs_sm[t]
        b = segs_sm[t + 1]
        w0 = (a // 8) * 8
        nch = jnp.where(b > a, (b - w0 + C - 1) // C, 0)

        def emit(ch, n2):
            plan_sm[n2] = t * 1024 + ch
            return n2 + 1

        return lax.fori_loop(0, nch, emit, n)

    ntot = lax.fori_loop(0, SEGS_PER_W, plan_segment, jnp.int32(0))

    def chunk_window(entry):
        t = entry // 1024
        ch = entry % 1024
        a = segs_sm[t]
        b = segs_sm[t + 1]
        wb = (a // 8) * 8 + ch * C
        cbase = jnp.minimum(wb, N_NODES - C)
        p = jnp.maximum(a, wb)
        q = jnp.minimum(wb + C, b)
        return t, cbase, p, q

    def start_fetch(i, buf, sem):
        @pl.when(i < ntot)
        def _():
            _, cbase, _, _ = chunk_window(plan_sm[i])
            pltpu.async_copy(node_hbm.at[pl.ds(cbase, C)], buf, sem)

    start_fetch(jnp.int32(0), rows0_v, sem0)
    start_fetch(jnp.int32(1), rows1_v, sem1)

    def accum_rows(buf, cbase, p, q, acc):
        n4 = (q - p) // 4

        def row4(i, acc2):
            l0 = p + i * 4 - cbase
            rows = [
                [buf[l0 + u, pl.ds(k * 16, 16)] for k in range(NVEC)]
                for u in range(4)
            ]
            return tuple(
                acc2[k] + ((rows[0][k] + rows[1][k]) + (rows[2][k] + rows[3][k]))
                for k in range(NVEC)
            )

        acc = lax.fori_loop(0, n4, row4, acc)

        def row1(r, acc2):
            lidx = r - cbase
            return tuple(
                acc2[k] + buf[lidx, pl.ds(k * 16, 16)]
                for k in range(NVEC)
            )

        return lax.fori_loop(p + n4 * 4, q, row1, acc)

    def half_step(i, buf, sem, state):
        t_prev, acc = state
        guard = i < ntot
        entry = plan_sm[jnp.minimum(i, jnp.maximum(ntot - 1, 0))]
        t, cbase, p, q = chunk_window(entry)
        q = jnp.where(guard, q, p)

        @pl.when(guard)
        def _():
            pltpu.make_async_copy(node_hbm.at[pl.ds(cbase, C)], buf, sem).wait()

        changed = guard & (t != t_prev)
        flush_row = jnp.where(changed, t_prev, jnp.int32(SEGS_PER_W))
        foff = flush_row * IN_DIM
        for k in range(NVEC):
            sums_v[pl.ds(foff + k * 16, 16)] = acc[k]

        keep = lax.broadcast_in_dim(
            jnp.where(changed, 0.0, 1.0).astype(jnp.float32), (16,), ())
        acc = tuple(a_k * keep for a_k in acc)
        acc = accum_rows(buf, cbase, p, q, acc)
        start_fetch(i + 2, buf, sem)
        t_prev = jnp.where(guard, t, t_prev)
        return t_prev, acc

    def pipelined_pair(h, state):
        state = half_step(h * 2, rows0_v, sem0, state)
        state = half_step(h * 2 + 1, rows1_v, sem1, state)
        return state

    t_last, acc = lax.fori_loop(0, (ntot + 1) // 2, pipelined_pair,
                                (jnp.int32(0), (zero16,) * NVEC))

    final_row = jnp.where(ntot > 0, t_last, jnp.int32(SEGS_PER_W))
    foff = final_row * IN_DIM
    for k in range(NVEC):
        sums_v[pl.ds(foff + k * 16, 16)] = acc[k]

    pltpu.sync_copy(sums_v.at[pl.ds(0, SEGS_PER_W * IN_DIM)],
                    sums_hbm.at[pl.ds(seg0 * IN_DIM, SEGS_PER_W * IN_DIM)])
    pltpu.sync_copy(cnts_v, cnts_hbm.at[pl.ds(seg0, SEGS_PER_W)])


_sc_segment_sum = pl.kernel(
    _sc_body,
    out_type=[
        jax.ShapeDtypeStruct((NUM_GRAPHS * IN_DIM,), jnp.float32),
        jax.ShapeDtypeStruct((NUM_GRAPHS, 16), jnp.float32),
    ],
    mesh=plsc.VectorSubcoreMesh(core_axis_name="c", subcore_axis_name="s",
                                num_cores=NC, num_subcores=NS),
    scratch_types=[
        pltpu.VMEM((C, IN_DIM), jnp.float32),
        pltpu.VMEM((C, IN_DIM), jnp.float32),
        pltpu.VMEM((NIDS_PAD,), jnp.int32),
        pltpu.VMEM(((SEGS_PER_W + 1) * IN_DIM,), jnp.float32),
        pltpu.VMEM((SEGS_PER_W, 16), jnp.float32),
        pltpu.SMEM((SEGS_PER_W + 1,), jnp.int32),
        pltpu.SMEM((416,), jnp.int32),
        pltpu.SemaphoreType.DMA,
        pltpu.SemaphoreType.DMA,
    ],
)


def _mlp_body(sums_ref, cnts_ref, w1_ref, b1_ref, w2_ref, b2_ref,
              g_ref, z_ref):
    counts = cnts_ref[:, 0]
    g = sums_ref[...] / jnp.maximum(counts, 1.0)[:, None]
    g_ref[...] = g
    h = lax.dot_general(g, w1_ref[...], (((1,), (1,)), ((), ())),
                        preferred_element_type=jnp.float32) + b1_ref[0, :]
    h = jnp.maximum(h, 0.0)
    z_ref[...] = lax.dot_general(h, w2_ref[...], (((1,), (1,)), ((), ())),
                                 preferred_element_type=jnp.float32) + b2_ref[0, :]


@jax.jit
def kernel(node_rep, batch_ids, W1, b1, W2, b2):
    ids = batch_ids.astype(jnp.int32)
    ids_pad = jnp.concatenate(
        [ids, jnp.full((NIDS_PAD - N_NODES,), NUM_GRAPHS, dtype=jnp.int32)])

    sums, cnts = _sc_segment_sum(node_rep, ids_pad)
    sums = sums.reshape(NUM_GRAPHS, IN_DIM)

    out = pl.pallas_call(
        _mlp_body,
        in_specs=[
            pl.BlockSpec((NUM_GRAPHS, IN_DIM), lambda: (0, 0)),
            pl.BlockSpec((NUM_GRAPHS, 16), lambda: (0, 0)),
            pl.BlockSpec((IN_DIM, IN_DIM), lambda: (0, 0)),
            pl.BlockSpec((1, IN_DIM), lambda: (0, 0)),
            pl.BlockSpec((IN_DIM, IN_DIM), lambda: (0, 0)),
            pl.BlockSpec((1, IN_DIM), lambda: (0, 0)),
        ],
        out_specs=[
            pl.BlockSpec((NUM_GRAPHS, IN_DIM), lambda: (0, 0)),
            pl.BlockSpec((NUM_GRAPHS, IN_DIM), lambda: (0, 0)),
        ],
        out_shape=[
            jax.ShapeDtypeStruct((NUM_GRAPHS, IN_DIM), jnp.float32),
            jax.ShapeDtypeStruct((NUM_GRAPHS, IN_DIM), jnp.float32),
        ],
    )(sums, cnts, W1, b1.reshape(1, IN_DIM), W2, b2.reshape(1, IN_DIM))
    return (out[0], out[1])
